# Initial kernel scaffold; baseline (speedup 1.0000x reference)
import jax
import jax.numpy as jnp
from jax.experimental import pallas as pl


def kernel(x, t, e_t, e_xct, Wl_tt, bl_tt, Wr_tt, Wl_x, bl_x, Wr_x, Wo, bo, gamma, beta, W1, b1, W2, b2):
    raise NotImplementedError("write your pallas kernel here")


# speedup vs baseline: 5.2669x; 5.2669x over previous
import functools

import jax
import jax.numpy as jnp
from jax import lax
from jax.experimental import pallas as pl
from jax.experimental.pallas import tpu as pltpu
from jax.experimental.pallas import tpu_sc as plsc

N = 10000
D = 128
E = 320000

NC = 2
NS = 16
L = 16

CHUNK = 128
CPT = 158
EPT = CPT * CHUNK
EP = NS * EPT
ROWS_PT = 640
NPAD = NS * ROWS_PT
CROWS = NPAD // D
ZR = 64
DSHIFT = 14
SMASK = (1 << DSHIFT) - 1


def _sc_body(t_hbm, x_hbm, pk_tt_hbm, pk_x_hbm,
             sum_tt_hbm, cnt_tt_hbm, sum_x_hbm, cnt_x_hbm,
             pkbuf, sidx_c, didx_c, rows_v, cnt_loc, cnt2d, ident_v, zbuf_v,
             acc_sh, cnt_sh, semg, sems, semc):
    cid = lax.axis_index("c")
    sid = lax.axis_index("s")

    zero16 = jnp.zeros((L,), jnp.float32)
    ones16 = jnp.ones((L,), jnp.float32)
    lane = lax.iota(jnp.int32, L)

    def _fill(r, _):
        for c in range(D // L):
            zbuf_v[r, pl.ds(c * L, L)] = zero16
        return 0
    lax.fori_loop(0, ZR, _fill, 0)

    def _zcnt(r, _):
        cnt_loc[pl.ds(r * L, L)] = zero16
        return 0
    lax.fori_loop(0, NPAD // L, _zcnt, 0)

    for k in range(CROWS // L):
        ident_v[pl.ds(k * L, L)] = lane + (k * L)

    for k in range(ROWS_PT // ZR):
        pltpu.sync_copy(zbuf_v, acc_sh.at[pl.ds(sid * ROWS_PT + k * ZR, ZR)])

    @pl.when(sid == 0)
    def _():
        pltpu.sync_copy(zbuf_v, cnt_sh.at[pl.ds(0, ZR)])
        pltpu.sync_copy(zbuf_v.at[pl.ds(0, CROWS - ZR)],
                        cnt_sh.at[pl.ds(ZR, CROWS - ZR)])

    plsc.subcore_barrier()

    def _edge_loop(table_hbm, pk3_hbm):
        def _chunk(j, _):
            pltpu.async_copy(pk3_hbm.at[sid, j], pkbuf, semc).wait()
            for k in range(CHUNK // L):
                s = pl.ds(k * L, L)
                v = pkbuf[s]
                sidx_c[s] = v & SMASK
                didx_c[s] = lax.shift_right_logical(v, DSHIFT)
            pltpu.async_copy(table_hbm.at[sidx_c], rows_v, semg).wait()
            pltpu.async_copy(rows_v, acc_sh.at[didx_c], sems, add=True).wait()
            for k in range(CHUNK // L):
                dv = didx_c[pl.ds(k * L, L)]
                plsc.addupdate_scatter(cnt_loc, [dv], ones16)
            return 0
        lax.fori_loop(0, CPT, _chunk, 0)

    @pl.when(cid == 0)
    def _():
        _edge_loop(t_hbm, pk_tt_hbm)

    @pl.when(cid == 1)
    def _():
        _edge_loop(x_hbm, pk_x_hbm)

    def _pack(r, _):
        for c in range(D // L):
            cnt2d[r, pl.ds(c * L, L)] = cnt_loc[pl.ds(r * D + c * L, L)]
        return 0
    lax.fori_loop(0, CROWS, _pack, 0)
    pltpu.async_copy(cnt2d, cnt_sh.at[ident_v], semc, add=True).wait()
    plsc.subcore_barrier()

    row = pl.ds(sid * ROWS_PT, ROWS_PT)

    @pl.when(cid == 0)
    def _():
        pltpu.sync_copy(acc_sh.at[row], sum_tt_hbm.at[row])

        @pl.when(sid == 0)
        def _():
            pltpu.sync_copy(cnt_sh, cnt_tt_hbm)

    @pl.when(cid == 1)
    def _():
        pltpu.sync_copy(acc_sh.at[row], sum_x_hbm.at[row])

        @pl.when(sid == 0)
        def _():
            pltpu.sync_copy(cnt_sh, cnt_x_hbm)


_sc_aggregate = functools.partial(
    pl.kernel,
    out_type=(
        jax.ShapeDtypeStruct((NPAD, D), jnp.float32),
        jax.ShapeDtypeStruct((CROWS, D), jnp.float32),
        jax.ShapeDtypeStruct((NPAD, D), jnp.float32),
        jax.ShapeDtypeStruct((CROWS, D), jnp.float32),
    ),
    mesh=plsc.VectorSubcoreMesh(core_axis_name="c", subcore_axis_name="s",
                                num_cores=NC, num_subcores=NS),
    compiler_params=pltpu.CompilerParams(needs_layout_passes=False),
    scratch_types=[
        pltpu.VMEM((CHUNK,), jnp.int32),
        pltpu.VMEM((CHUNK,), jnp.int32),
        pltpu.VMEM((CHUNK,), jnp.int32),
        pltpu.VMEM((CHUNK, D), jnp.float32),
        pltpu.VMEM((NPAD,), jnp.float32),
        pltpu.VMEM((CROWS, D), jnp.float32),
        pltpu.VMEM((CROWS,), jnp.int32),
        pltpu.VMEM((ZR, D), jnp.float32),
        pltpu.VMEM_SHARED((NPAD, D), jnp.float32),
        pltpu.VMEM_SHARED((CROWS, D), jnp.float32),
        pltpu.SemaphoreType.DMA,
        pltpu.SemaphoreType.DMA,
        pltpu.SemaphoreType.DMA,
    ],
)(_sc_body)


def _tc_body(t_ref, x_ref, stt_ref, ctt_ref, sx_ref, cx_ref,
             wltt_ref, wrtt_ref, wlx_ref, wrx_ref, wo_ref, w1_ref, w2_ref,
             bltt_ref, blx_ref, bo_ref, gamma_ref, beta_ref, b1_ref, b2_ref,
             out_x_ref, out_t_ref):
    f32 = jnp.float32
    tb = t_ref[...]
    xb = x_ref[...]
    agg_tt = stt_ref[...] / jnp.maximum(ctt_ref[...], 1.0)
    agg_x = sx_ref[...] / jnp.maximum(cx_ref[...], 1.0)
    h = (tb
         + jnp.dot(agg_tt, wltt_ref[...], preferred_element_type=f32)
         + bltt_ref[...]
         + jnp.dot(tb, wrtt_ref[...], preferred_element_type=f32)
         + jnp.dot(agg_x, wlx_ref[...], preferred_element_type=f32)
         + blx_ref[...]
         + jnp.dot(tb, wrx_ref[...], preferred_element_type=f32))
    t2 = jnp.maximum(h, 0.0)
    o = jnp.maximum(jnp.dot(t2, wo_ref[...], preferred_element_type=f32)
                    + bo_ref[...], 0.0)
    mu = jnp.mean(o, axis=-1, keepdims=True)
    cen = o - mu
    var = jnp.mean(cen * cen, axis=-1, keepdims=True)
    ln = cen * lax.rsqrt(var + 1e-5) * gamma_ref[...] + beta_ref[...]
    out_t_ref[...] = t2 + ln
    fcx = jnp.dot(
        jnp.maximum(jnp.dot(xb, w1_ref[...], preferred_element_type=f32)
                    + b1_ref[...], 0.0),
        w2_ref[...], preferred_element_type=f32) + b2_ref[...]
    out_x_ref[...] = xb + fcx


def kernel(x, t, e_t, e_xct, Wl_tt, bl_tt, Wr_tt, Wl_x, bl_x, Wr_x,
           Wo, bo, gamma, beta, W1, b1, W2, b2):
    def _prep(e):
        src = jnp.pad(e[0], (0, EP - E))
        dst = jnp.pad(e[1], (0, EP - E),
                      constant_values=N)
        return (src | (dst << DSHIFT)).reshape(NS, CPT, CHUNK)

    pk_tt = _prep(e_t)
    pk_x = _prep(e_xct)

    sum_tt, cnt_tt, sum_x, cnt_x = _sc_aggregate(t, x, pk_tt, pk_x)

    R = 2000
    row_blk = pl.BlockSpec((R, D), lambda i: (i, 0))
    cnt_blk = pl.BlockSpec((R, 1), lambda i: (i, 0))
    w_blk = pl.BlockSpec((D, D), lambda i: (0, 0))
    v_blk = pl.BlockSpec((1, D), lambda i: (0, 0))

    out_x, out_t = pl.pallas_call(
        _tc_body,
        grid=(N // R,),
        in_specs=[row_blk, row_blk, row_blk, cnt_blk, row_blk, cnt_blk,
                  w_blk, w_blk, w_blk, w_blk, w_blk, w_blk, w_blk,
                  v_blk, v_blk, v_blk, v_blk, v_blk, v_blk, v_blk],
        out_specs=[row_blk, row_blk],
        out_shape=[jax.ShapeDtypeStruct((N, D), jnp.float32),
                   jax.ShapeDtypeStruct((N, D), jnp.float32)],
    )(t, x, sum_tt, cnt_tt.reshape(NPAD, 1), sum_x, cnt_x.reshape(NPAD, 1),
      Wl_tt, Wr_tt, Wl_x, Wr_x, Wo, W1, W2,
      bl_tt.reshape(1, D), bl_x.reshape(1, D), bo.reshape(1, D),
      gamma.reshape(1, D), beta.reshape(1, D),
      b1.reshape(1, D), b2.reshape(1, D))

    return (out_x, out_t)


# speedup vs baseline: 7.8482x; 1.4901x over previous
import functools

import jax
import jax.numpy as jnp
from jax import lax
from jax.experimental import pallas as pl
from jax.experimental.pallas import tpu as pltpu
from jax.experimental.pallas import tpu_sc as plsc

N = 10000
D = 128
E = 320000

NC = 2
NS = 16
L = 16

CHUNK = 128
NB = 2
CPT = 158
EPT = CPT * CHUNK
EP = NS * EPT
ROWS_PT = 640
NPAD = NS * ROWS_PT
CROWS = NPAD // D
ZR = 16
DSHIFT = 14
SMASK = (1 << DSHIFT) - 1


def _sc_body(t_hbm, x_hbm, pk_tt_hbm, pk_x_hbm,
             sum_tt_hbm, cnt_tt_hbm, sum_x_hbm, cnt_x_hbm, *scr):
    pk = scr[0:NB]
    sidx = scr[NB:2 * NB]
    didx = scr[2 * NB:3 * NB]
    rows = scr[3 * NB:4 * NB]
    cnt2d, ident_v, zbuf_v, acc_sh, cnt_sh = scr[4 * NB:4 * NB + 5]
    semi = scr[4 * NB + 5:5 * NB + 5]
    semg = scr[5 * NB + 5:6 * NB + 5]
    sems = scr[6 * NB + 5:7 * NB + 5]
    semz = scr[7 * NB + 5]

    cid = lax.axis_index("c")
    sid = lax.axis_index("s")

    zero16 = jnp.zeros((L,), jnp.float32)
    ones16 = jnp.ones((L,), jnp.float32)
    lane = lax.iota(jnp.int32, L)

    def _fill(r, _):
        for c in range(D // L):
            zbuf_v[r, pl.ds(c * L, L)] = zero16
        return 0
    lax.fori_loop(0, ZR, _fill, 0)

    def _zcnt(r, _):
        for c in range(D // L):
            cnt2d[r, pl.ds(c * L, L)] = zero16
        return 0
    lax.fori_loop(0, CROWS, _zcnt, 0)

    for k in range(CROWS // L):
        ident_v[pl.ds(k * L, L)] = lane + (k * L)

    nz = ROWS_PT // ZR
    for k in range(nz):
        pltpu.async_copy(zbuf_v, acc_sh.at[pl.ds(sid * ROWS_PT + k * ZR, ZR)],
                         semz)
    for k in range(nz):
        pltpu.make_async_copy(
            zbuf_v, acc_sh.at[pl.ds(sid * ROWS_PT + k * ZR, ZR)], semz).wait()

    @pl.when(sid == 0)
    def _():
        for k in range(CROWS // ZR):
            pltpu.sync_copy(zbuf_v, cnt_sh.at[pl.ds(k * ZR, ZR)])
        rem = CROWS - (CROWS // ZR) * ZR
        if rem:
            pltpu.sync_copy(zbuf_v.at[pl.ds(0, rem)],
                            cnt_sh.at[pl.ds(CROWS - rem, rem)])

    plsc.subcore_barrier()

    def _edge_loop(table_hbm, pk3_hbm):
        def _issue_idx(j, s):
            pltpu.async_copy(pk3_hbm.at[sid, j], pk[s], semi[s])

        def _wait_idx(j, s):
            pltpu.make_async_copy(pk3_hbm.at[sid, j], pk[s], semi[s]).wait()

        def _unpack(s):
            for k in range(CHUNK // L):
                sl = pl.ds(k * L, L)
                v = pk[s][sl]
                sidx[s][sl] = v & SMASK
                didx[s][sl] = lax.shift_right_logical(v, DSHIFT)

        def _issue_gather(s):
            pltpu.async_copy(table_hbm.at[sidx[s]], rows[s], semg[s])

        def _wait_gather(s):
            pltpu.make_async_copy(table_hbm.at[sidx[s]], rows[s],
                                  semg[s]).wait()

        def _issue_scatter(s):
            pltpu.async_copy(rows[s], acc_sh.at[didx[s]], sems[s], add=True)

        def _wait_scatter(s):
            pltpu.make_async_copy(rows[s], acc_sh.at[didx[s]], sems[s]).wait()

        def _counts(s):
            for k in range(CHUNK // L):
                dv = didx[s][pl.ds(k * L, L)]
                plsc.addupdate_scatter(
                    cnt2d, [lax.shift_right_logical(dv, 7), dv & (D - 1)],
                    ones16)

        for b in range(NB):
            _issue_idx(b, b)
        _wait_idx(0, 0)
        _unpack(0)
        _issue_gather(0)
        _issue_idx(NB, 0)

        def _outer(g, _):
            for b in range(NB):
                s = b
                s1 = (b + 1) % NB
                j = g * NB + b

                @pl.when(j - (NB - 1) >= 0)
                def _():
                    _wait_scatter(s1)

                @pl.when(j + 1 <= CPT - 1)
                def _():
                    _wait_idx(j + 1, s1)
                    _unpack(s1)
                    _issue_gather(s1)

                _wait_gather(s)
                _issue_scatter(s)
                _counts(s)

                @pl.when(j + NB + 1 <= CPT - 1)
                def _():
                    _issue_idx(j + NB + 1, s1)
            return 0
        lax.fori_loop(0, CPT // NB, _outer, 0)

        for b in range(NB - 1):
            _wait_scatter((CPT - (NB - 1) + b) % NB)

    @pl.when(cid == 0)
    def _():
        _edge_loop(t_hbm, pk_tt_hbm)

    @pl.when(cid == 1)
    def _():
        _edge_loop(x_hbm, pk_x_hbm)

    pltpu.async_copy(cnt2d, cnt_sh.at[ident_v], semz, add=True).wait()
    plsc.subcore_barrier()

    row = pl.ds(sid * ROWS_PT, ROWS_PT)

    @pl.when(cid == 0)
    def _():
        pltpu.sync_copy(acc_sh.at[row], sum_tt_hbm.at[row])

        @pl.when(sid == 0)
        def _():
            pltpu.sync_copy(cnt_sh, cnt_tt_hbm)

    @pl.when(cid == 1)
    def _():
        pltpu.sync_copy(acc_sh.at[row], sum_x_hbm.at[row])

        @pl.when(sid == 0)
        def _():
            pltpu.sync_copy(cnt_sh, cnt_x_hbm)


_sc_aggregate = functools.partial(
    pl.kernel,
    out_type=(
        jax.ShapeDtypeStruct((NPAD, D), jnp.float32),
        jax.ShapeDtypeStruct((CROWS, D), jnp.float32),
        jax.ShapeDtypeStruct((NPAD, D), jnp.float32),
        jax.ShapeDtypeStruct((CROWS, D), jnp.float32),
    ),
    mesh=plsc.VectorSubcoreMesh(core_axis_name="c", subcore_axis_name="s",
                                num_cores=NC, num_subcores=NS),
    compiler_params=pltpu.CompilerParams(needs_layout_passes=False),
    scratch_types=(
        [pltpu.VMEM((CHUNK,), jnp.int32)] * NB
        + [pltpu.VMEM((CHUNK,), jnp.int32)] * NB
        + [pltpu.VMEM((CHUNK,), jnp.int32)] * NB
        + [pltpu.VMEM((CHUNK, D), jnp.float32)] * NB
        + [pltpu.VMEM((CROWS, D), jnp.float32),
           pltpu.VMEM((CROWS,), jnp.int32),
           pltpu.VMEM((ZR, D), jnp.float32),
           pltpu.VMEM_SHARED((NPAD, D), jnp.float32),
           pltpu.VMEM_SHARED((CROWS, D), jnp.float32)]
        + [pltpu.SemaphoreType.DMA] * (3 * NB + 1)
    ),
)(_sc_body)


def _tc_body(t_ref, x_ref, stt_ref, ctt_ref, sx_ref, cx_ref,
             wltt_ref, wrtt_ref, wlx_ref, wrx_ref, wo_ref, w1_ref, w2_ref,
             bltt_ref, blx_ref, bo_ref, gamma_ref, beta_ref, b1_ref, b2_ref,
             out_x_ref, out_t_ref):
    f32 = jnp.float32
    tb = t_ref[...]
    xb = x_ref[...]
    agg_tt = stt_ref[...] / jnp.maximum(ctt_ref[...], 1.0)
    agg_x = sx_ref[...] / jnp.maximum(cx_ref[...], 1.0)
    h = (tb
         + jnp.dot(agg_tt, wltt_ref[...], preferred_element_type=f32)
         + bltt_ref[...]
         + jnp.dot(tb, wrtt_ref[...], preferred_element_type=f32)
         + jnp.dot(agg_x, wlx_ref[...], preferred_element_type=f32)
         + blx_ref[...]
         + jnp.dot(tb, wrx_ref[...], preferred_element_type=f32))
    t2 = jnp.maximum(h, 0.0)
    o = jnp.maximum(jnp.dot(t2, wo_ref[...], preferred_element_type=f32)
                    + bo_ref[...], 0.0)
    mu = jnp.mean(o, axis=-1, keepdims=True)
    cen = o - mu
    var = jnp.mean(cen * cen, axis=-1, keepdims=True)
    ln = cen * lax.rsqrt(var + 1e-5) * gamma_ref[...] + beta_ref[...]
    out_t_ref[...] = t2 + ln
    fcx = jnp.dot(
        jnp.maximum(jnp.dot(xb, w1_ref[...], preferred_element_type=f32)
                    + b1_ref[...], 0.0),
        w2_ref[...], preferred_element_type=f32) + b2_ref[...]
    out_x_ref[...] = xb + fcx


def kernel(x, t, e_t, e_xct, Wl_tt, bl_tt, Wr_tt, Wl_x, bl_x, Wr_x,
           Wo, bo, gamma, beta, W1, b1, W2, b2):
    def _prep(e):
        src = jnp.pad(e[0], (0, EP - E))
        dst = jnp.pad(e[1], (0, EP - E),
                      constant_values=N)
        return (src | (dst << DSHIFT)).reshape(NS, CPT, CHUNK)

    pk_tt = _prep(e_t)
    pk_x = _prep(e_xct)

    sum_tt, cnt_tt, sum_x, cnt_x = _sc_aggregate(t, x, pk_tt, pk_x)

    R = 2000
    row_blk = pl.BlockSpec((R, D), lambda i: (i, 0))
    cnt_blk = pl.BlockSpec((R, 1), lambda i: (i, 0))
    w_blk = pl.BlockSpec((D, D), lambda i: (0, 0))
    v_blk = pl.BlockSpec((1, D), lambda i: (0, 0))

    out_x, out_t = pl.pallas_call(
        _tc_body,
        grid=(N // R,),
        in_specs=[row_blk, row_blk, row_blk, cnt_blk, row_blk, cnt_blk,
                  w_blk, w_blk, w_blk, w_blk, w_blk, w_blk, w_blk,
                  v_blk, v_blk, v_blk, v_blk, v_blk, v_blk, v_blk],
        out_specs=[row_blk, row_blk],
        out_shape=[jax.ShapeDtypeStruct((N, D), jnp.float32),
                   jax.ShapeDtypeStruct((N, D), jnp.float32)],
    )(t, x, sum_tt, cnt_tt.reshape(NPAD, 1), sum_x, cnt_x.reshape(NPAD, 1),
      Wl_tt, Wr_tt, Wl_x, Wr_x, Wo, W1, W2,
      bl_tt.reshape(1, D), bl_x.reshape(1, D), bo.reshape(1, D),
      gamma.reshape(1, D), beta.reshape(1, D),
      b1.reshape(1, D), b2.reshape(1, D))

    return (out_x, out_t)


# speedup vs baseline: 7.8983x; 1.0064x over previous
import functools

import jax
import jax.numpy as jnp
from jax import lax
from jax.experimental import pallas as pl
from jax.experimental.pallas import tpu as pltpu
from jax.experimental.pallas import tpu_sc as plsc

N = 10000
D = 128
E = 320000

NC = 2
NS = 16
L = 16

CHUNK = 128
NB = 2
CPT = 158
EPT = CPT * CHUNK
EP = NS * EPT
ROWS_PT = 640
NPAD = NS * ROWS_PT
CROWS = NPAD // D
ZR = 16
DSHIFT = 14
SMASK = (1 << DSHIFT) - 1


def _sc_body(t_hbm, x_hbm, pk_tt_hbm, pk_x_hbm,
             sum_tt_hbm, cnt_tt_hbm, sum_x_hbm, cnt_x_hbm, *scr):
    pk = scr[0:NB]
    sidx = scr[NB:2 * NB]
    didx = scr[2 * NB:3 * NB]
    rows = scr[3 * NB:4 * NB]
    cnt2d, ident_v, zbuf_v, acc_sh, cnt_sh = scr[4 * NB:4 * NB + 5]
    semi = scr[4 * NB + 5:5 * NB + 5]
    semg = scr[5 * NB + 5:6 * NB + 5]
    sems = scr[6 * NB + 5:7 * NB + 5]
    semz = scr[7 * NB + 5]

    cid = lax.axis_index("c")
    sid = lax.axis_index("s")

    zero16 = jnp.zeros((L,), jnp.float32)
    ones16 = jnp.ones((L,), jnp.float32)
    lane = lax.iota(jnp.int32, L)

    def _fill(r, _):
        for c in range(D // L):
            zbuf_v[r, pl.ds(c * L, L)] = zero16
        return 0
    lax.fori_loop(0, ZR, _fill, 0)

    def _zcnt(r, _):
        for c in range(D // L):
            cnt2d[r, pl.ds(c * L, L)] = zero16
        return 0
    lax.fori_loop(0, CROWS, _zcnt, 0)

    for k in range(CROWS // L):
        ident_v[pl.ds(k * L, L)] = lane + (k * L)

    nz = ROWS_PT // ZR
    for k in range(nz):
        pltpu.async_copy(zbuf_v, acc_sh.at[pl.ds(sid * ROWS_PT + k * ZR, ZR)],
                         semz)
    for k in range(nz):
        pltpu.make_async_copy(
            zbuf_v, acc_sh.at[pl.ds(sid * ROWS_PT + k * ZR, ZR)], semz).wait()

    @pl.when(sid == 0)
    def _():
        for k in range(CROWS // ZR):
            pltpu.sync_copy(zbuf_v, cnt_sh.at[pl.ds(k * ZR, ZR)])
        rem = CROWS - (CROWS // ZR) * ZR
        if rem:
            pltpu.sync_copy(zbuf_v.at[pl.ds(0, rem)],
                            cnt_sh.at[pl.ds(CROWS - rem, rem)])

    plsc.subcore_barrier()

    def _edge_loop(table_hbm, pk3_hbm):
        def _issue_idx(j, s):
            pltpu.async_copy(pk3_hbm.at[sid, j], pk[s], semi[s])

        def _wait_idx(j, s):
            pltpu.make_async_copy(pk3_hbm.at[sid, j], pk[s], semi[s]).wait()

        def _unpack(s):
            for k in range(CHUNK // L):
                sl = pl.ds(k * L, L)
                v = pk[s][sl]
                sidx[s][sl] = v & SMASK
                didx[s][sl] = lax.shift_right_logical(v, DSHIFT)

        def _issue_gather(s):
            pltpu.async_copy(table_hbm.at[sidx[s]], rows[s], semg[s])

        def _wait_gather(s):
            pltpu.make_async_copy(table_hbm.at[sidx[s]], rows[s],
                                  semg[s]).wait()

        def _issue_scatter(s):
            pltpu.async_copy(rows[s], acc_sh.at[didx[s]], sems[s], add=True)

        def _wait_scatter(s):
            pltpu.make_async_copy(rows[s], acc_sh.at[didx[s]], sems[s]).wait()

        def _counts(s):
            for k in range(CHUNK // L):
                dv = didx[s][pl.ds(k * L, L)]
                plsc.addupdate_scatter(
                    cnt2d, [lax.shift_right_logical(dv, 7), dv & (D - 1)],
                    ones16)

        for b in range(NB):
            _issue_idx(b, b)
        _wait_idx(0, 0)
        _unpack(0)
        _issue_gather(0)
        _issue_idx(NB, 0)

        def _outer(g, _):
            for b in range(NB):
                s = b
                s1 = (b + 1) % NB
                j = g * NB + b

                @pl.when(j - (NB - 1) >= 0)
                def _():
                    _wait_scatter(s1)

                @pl.when(j + 1 <= CPT - 1)
                def _():
                    _wait_idx(j + 1, s1)
                    _unpack(s1)
                    _issue_gather(s1)

                _wait_gather(s)
                _issue_scatter(s)
                _counts(s)

                @pl.when(j + NB + 1 <= CPT - 1)
                def _():
                    _issue_idx(j + NB + 1, s1)
            return 0
        lax.fori_loop(0, CPT // NB, _outer, 0)

        for b in range(NB - 1):
            _wait_scatter((CPT - (NB - 1) + b) % NB)

    @pl.when(cid == 0)
    def _():
        _edge_loop(t_hbm, pk_tt_hbm)

    @pl.when(cid == 1)
    def _():
        _edge_loop(x_hbm, pk_x_hbm)

    pltpu.async_copy(cnt2d, cnt_sh.at[ident_v], semz, add=True).wait()
    plsc.subcore_barrier()

    row = pl.ds(sid * ROWS_PT, ROWS_PT)

    @pl.when(cid == 0)
    def _():
        pltpu.sync_copy(acc_sh.at[row], sum_tt_hbm.at[row])

        @pl.when(sid == 0)
        def _():
            pltpu.sync_copy(cnt_sh, cnt_tt_hbm)

    @pl.when(cid == 1)
    def _():
        pltpu.sync_copy(acc_sh.at[row], sum_x_hbm.at[row])

        @pl.when(sid == 0)
        def _():
            pltpu.sync_copy(cnt_sh, cnt_x_hbm)


_sc_aggregate = functools.partial(
    pl.kernel,
    out_type=(
        jax.ShapeDtypeStruct((NPAD, D), jnp.float32),
        jax.ShapeDtypeStruct((CROWS, D), jnp.float32),
        jax.ShapeDtypeStruct((NPAD, D), jnp.float32),
        jax.ShapeDtypeStruct((CROWS, D), jnp.float32),
    ),
    mesh=plsc.VectorSubcoreMesh(core_axis_name="c", subcore_axis_name="s",
                                num_cores=NC, num_subcores=NS),
    compiler_params=pltpu.CompilerParams(needs_layout_passes=False),
    scratch_types=(
        [pltpu.VMEM((CHUNK,), jnp.int32)] * NB
        + [pltpu.VMEM((CHUNK,), jnp.int32)] * NB
        + [pltpu.VMEM((CHUNK,), jnp.int32)] * NB
        + [pltpu.VMEM((CHUNK, D), jnp.float32)] * NB
        + [pltpu.VMEM((CROWS, D), jnp.float32),
           pltpu.VMEM((CROWS,), jnp.int32),
           pltpu.VMEM((ZR, D), jnp.float32),
           pltpu.VMEM_SHARED((NPAD, D), jnp.float32),
           pltpu.VMEM_SHARED((CROWS, D), jnp.float32)]
        + [pltpu.SemaphoreType.DMA] * (3 * NB + 1)
    ),
)(_sc_body)


def _tc_pre_body(t_ref, x_ref, wrtt_ref, wrx_ref, w1_ref, w2_ref,
                 bltt_ref, blx_ref, b1_ref, b2_ref,
                 out_x_ref, tpart_ref):
    f32 = jnp.float32
    tb = t_ref[...]
    xb = x_ref[...]
    tpart_ref[...] = (tb
                      + jnp.dot(tb, wrtt_ref[...], preferred_element_type=f32)
                      + bltt_ref[...]
                      + jnp.dot(tb, wrx_ref[...], preferred_element_type=f32)
                      + blx_ref[...])
    fcx = jnp.dot(
        jnp.maximum(jnp.dot(xb, w1_ref[...], preferred_element_type=f32)
                    + b1_ref[...], 0.0),
        w2_ref[...], preferred_element_type=f32) + b2_ref[...]
    out_x_ref[...] = xb + fcx


def _tc_post_body(tpart_ref, stt_ref, ctt_ref, sx_ref, cx_ref,
                  wltt_ref, wlx_ref, wo_ref,
                  bo_ref, gamma_ref, beta_ref,
                  out_t_ref):
    f32 = jnp.float32
    agg_tt = stt_ref[...] / jnp.maximum(ctt_ref[...], 1.0)
    agg_x = sx_ref[...] / jnp.maximum(cx_ref[...], 1.0)
    h = (tpart_ref[...]
         + jnp.dot(agg_tt, wltt_ref[...], preferred_element_type=f32)
         + jnp.dot(agg_x, wlx_ref[...], preferred_element_type=f32))
    t2 = jnp.maximum(h, 0.0)
    o = jnp.maximum(jnp.dot(t2, wo_ref[...], preferred_element_type=f32)
                    + bo_ref[...], 0.0)
    mu = jnp.mean(o, axis=-1, keepdims=True)
    cen = o - mu
    var = jnp.mean(cen * cen, axis=-1, keepdims=True)
    ln = cen * lax.rsqrt(var + 1e-5) * gamma_ref[...] + beta_ref[...]
    out_t_ref[...] = t2 + ln


def kernel(x, t, e_t, e_xct, Wl_tt, bl_tt, Wr_tt, Wl_x, bl_x, Wr_x,
           Wo, bo, gamma, beta, W1, b1, W2, b2):
    def _prep(e):
        src = jnp.pad(e[0], (0, EP - E))
        dst = jnp.pad(e[1], (0, EP - E),
                      constant_values=N)
        return (src | (dst << DSHIFT)).reshape(NS, CPT, CHUNK)

    pk_tt = _prep(e_t)
    pk_x = _prep(e_xct)

    sum_tt, cnt_tt, sum_x, cnt_x = _sc_aggregate(t, x, pk_tt, pk_x)

    R = 2000
    row_blk = pl.BlockSpec((R, D), lambda i: (i, 0))
    cnt_blk = pl.BlockSpec((R, 1), lambda i: (i, 0))
    w_blk = pl.BlockSpec((D, D), lambda i: (0, 0))
    v_blk = pl.BlockSpec((1, D), lambda i: (0, 0))

    out_x, tpart = pl.pallas_call(
        _tc_pre_body,
        grid=(N // R,),
        in_specs=[row_blk, row_blk, w_blk, w_blk, w_blk, w_blk,
                  v_blk, v_blk, v_blk, v_blk],
        out_specs=[row_blk, row_blk],
        out_shape=[jax.ShapeDtypeStruct((N, D), jnp.float32),
                   jax.ShapeDtypeStruct((N, D), jnp.float32)],
    )(t, x, Wr_tt, Wr_x, W1, W2,
      bl_tt.reshape(1, D), bl_x.reshape(1, D),
      b1.reshape(1, D), b2.reshape(1, D))

    out_t = pl.pallas_call(
        _tc_post_body,
        grid=(N // R,),
        in_specs=[row_blk, row_blk, cnt_blk, row_blk, cnt_blk,
                  w_blk, w_blk, w_blk,
                  v_blk, v_blk, v_blk],
        out_specs=row_blk,
        out_shape=jax.ShapeDtypeStruct((N, D), jnp.float32),
    )(tpart, sum_tt, cnt_tt.reshape(NPAD, 1), sum_x, cnt_x.reshape(NPAD, 1),
      Wl_tt, Wl_x, Wo,
      bo.reshape(1, D), gamma.reshape(1, D), beta.reshape(1, D))

    return (out_x, out_t)


# speedup vs baseline: 8.3009x; 1.0510x over previous
import functools

import jax
import jax.numpy as jnp
from jax import lax
from jax.experimental import pallas as pl
from jax.experimental.pallas import tpu as pltpu
from jax.experimental.pallas import tpu_sc as plsc

N = 10000
D = 128
E = 320000

NC = 2
NS = 16
L = 16

CHUNK = 128
NB = 4
CPT = 160
EPT = CPT * CHUNK
EP = NS * EPT
ROWS_PT = 640
NPAD = NS * ROWS_PT
CROWS = NPAD // D
ZR = 16
DSHIFT = 14
SMASK = (1 << DSHIFT) - 1


def _sc_body(t_hbm, x_hbm, pk_tt_hbm, pk_x_hbm,
             sum_tt_hbm, cnt_tt_hbm, sum_x_hbm, cnt_x_hbm, *scr):
    pk = scr[0:NB]
    sidx = scr[NB:2 * NB]
    didx = scr[2 * NB:3 * NB]
    rows = scr[3 * NB:4 * NB]
    cnt2d, ident_v, zbuf_v, zbuf16_v, acc_sh, cnt_sh = scr[4 * NB:4 * NB + 6]
    semi = scr[4 * NB + 6:5 * NB + 6]
    semg = scr[5 * NB + 6:6 * NB + 6]
    sems = scr[6 * NB + 6:7 * NB + 6]
    semz = scr[7 * NB + 6]

    cid = lax.axis_index("c")
    sid = lax.axis_index("s")

    zero16 = jnp.zeros((L,), jnp.float32)
    ones16 = jnp.ones((L,), jnp.float32)
    lane = lax.iota(jnp.int32, L)

    zero32b = jnp.zeros((2 * L,), jnp.bfloat16)

    def _fill(r, _):
        for c in range(D // L):
            zbuf_v[r, pl.ds(c * L, L)] = zero16
        for c in range(D // (2 * L)):
            zbuf16_v[r, pl.ds(c * 2 * L, 2 * L)] = zero32b
        return 0
    lax.fori_loop(0, ZR, _fill, 0)

    def _zcnt(r, _):
        for c in range(D // L):
            cnt2d[r, pl.ds(c * L, L)] = zero16
        return 0
    lax.fori_loop(0, CROWS, _zcnt, 0)

    for k in range(CROWS // L):
        ident_v[pl.ds(k * L, L)] = lane + (k * L)

    nz = ROWS_PT // ZR
    for k in range(nz):
        pltpu.async_copy(zbuf16_v,
                         acc_sh.at[pl.ds(sid * ROWS_PT + k * ZR, ZR)], semz)
    for k in range(nz):
        pltpu.make_async_copy(
            zbuf16_v, acc_sh.at[pl.ds(sid * ROWS_PT + k * ZR, ZR)],
            semz).wait()

    @pl.when(sid == 0)
    def _():
        for k in range(CROWS // ZR):
            pltpu.sync_copy(zbuf_v, cnt_sh.at[pl.ds(k * ZR, ZR)])
        rem = CROWS - (CROWS // ZR) * ZR
        if rem:
            pltpu.sync_copy(zbuf_v.at[pl.ds(0, rem)],
                            cnt_sh.at[pl.ds(CROWS - rem, rem)])

    plsc.subcore_barrier()

    def _edge_loop(table_hbm, pk3_hbm):
        def _issue_idx(j, s):
            pltpu.async_copy(pk3_hbm.at[sid, j], pk[s], semi[s])

        def _wait_idx(j, s):
            pltpu.make_async_copy(pk3_hbm.at[sid, j], pk[s], semi[s]).wait()

        def _unpack(s):
            for k in range(CHUNK // L):
                sl = pl.ds(k * L, L)
                v = pk[s][sl]
                sidx[s][sl] = v & SMASK
                didx[s][sl] = lax.shift_right_logical(v, DSHIFT)

        def _issue_gather(s):
            pltpu.async_copy(table_hbm.at[sidx[s]], rows[s], semg[s])

        def _wait_gather(s):
            pltpu.make_async_copy(table_hbm.at[sidx[s]], rows[s],
                                  semg[s]).wait()

        def _issue_scatter(s):
            pltpu.async_copy(rows[s], acc_sh.at[didx[s]], sems[s], add=True)

        def _wait_scatter(s):
            pltpu.make_async_copy(rows[s], acc_sh.at[didx[s]], sems[s]).wait()

        def _counts(s):
            for k in range(CHUNK // L):
                dv = didx[s][pl.ds(k * L, L)]
                plsc.addupdate_scatter(
                    cnt2d, [lax.shift_right_logical(dv, 7), dv & (D - 1)],
                    ones16)

        for b in range(NB):
            _issue_idx(b, b)
        _wait_idx(0, 0)
        _unpack(0)
        _issue_gather(0)
        _issue_idx(NB, 0)

        def _outer(g, _):
            for b in range(NB):
                s = b
                s1 = (b + 1) % NB
                j = g * NB + b

                @pl.when(j - (NB - 1) >= 0)
                def _():
                    _wait_scatter(s1)

                @pl.when(j + 1 <= CPT - 1)
                def _():
                    _wait_idx(j + 1, s1)
                    _unpack(s1)
                    _issue_gather(s1)

                _wait_gather(s)
                _issue_scatter(s)
                _counts(s)

                @pl.when(j + NB + 1 <= CPT - 1)
                def _():
                    _issue_idx(j + NB + 1, s1)
            return 0
        lax.fori_loop(0, CPT // NB, _outer, 0)

        for b in range(NB - 1):
            _wait_scatter((CPT - (NB - 1) + b) % NB)

    @pl.when(cid == 0)
    def _():
        _edge_loop(t_hbm, pk_tt_hbm)

    @pl.when(cid == 1)
    def _():
        _edge_loop(x_hbm, pk_x_hbm)

    pltpu.async_copy(cnt2d, cnt_sh.at[ident_v], semz, add=True).wait()
    plsc.subcore_barrier()

    row = pl.ds(sid * ROWS_PT, ROWS_PT)

    @pl.when(cid == 0)
    def _():
        pltpu.sync_copy(acc_sh.at[row], sum_tt_hbm.at[row])

        @pl.when(sid == 0)
        def _():
            pltpu.sync_copy(cnt_sh, cnt_tt_hbm)

    @pl.when(cid == 1)
    def _():
        pltpu.sync_copy(acc_sh.at[row], sum_x_hbm.at[row])

        @pl.when(sid == 0)
        def _():
            pltpu.sync_copy(cnt_sh, cnt_x_hbm)


_sc_aggregate = functools.partial(
    pl.kernel,
    out_type=(
        jax.ShapeDtypeStruct((NPAD, D), jnp.bfloat16),
        jax.ShapeDtypeStruct((CROWS, D), jnp.float32),
        jax.ShapeDtypeStruct((NPAD, D), jnp.bfloat16),
        jax.ShapeDtypeStruct((CROWS, D), jnp.float32),
    ),
    mesh=plsc.VectorSubcoreMesh(core_axis_name="c", subcore_axis_name="s",
                                num_cores=NC, num_subcores=NS),
    compiler_params=pltpu.CompilerParams(needs_layout_passes=False,
                                         use_tc_tiling_on_sc=False),
    scratch_types=(
        [pltpu.VMEM((CHUNK,), jnp.int32)] * NB
        + [pltpu.VMEM((CHUNK,), jnp.int32)] * NB
        + [pltpu.VMEM((CHUNK,), jnp.int32)] * NB
        + [pltpu.VMEM((CHUNK, D), jnp.bfloat16)] * NB
        + [pltpu.VMEM((CROWS, D), jnp.float32),
           pltpu.VMEM((CROWS,), jnp.int32),
           pltpu.VMEM((ZR, D), jnp.float32),
           pltpu.VMEM((ZR, D), jnp.bfloat16),
           pltpu.VMEM_SHARED((NPAD, D), jnp.bfloat16),
           pltpu.VMEM_SHARED((CROWS, D), jnp.float32)]
        + [pltpu.SemaphoreType.DMA] * (3 * NB + 1)
    ),
)(_sc_body)


def _tc_pre_body(t_ref, x_ref, wrtt_ref, wrx_ref, w1_ref, w2_ref,
                 bltt_ref, blx_ref, b1_ref, b2_ref,
                 out_x_ref, tpart_ref):
    f32 = jnp.float32
    tb = t_ref[...]
    xb = x_ref[...]
    tpart_ref[...] = (tb
                      + jnp.dot(tb, wrtt_ref[...], preferred_element_type=f32)
                      + bltt_ref[...]
                      + jnp.dot(tb, wrx_ref[...], preferred_element_type=f32)
                      + blx_ref[...])
    fcx = jnp.dot(
        jnp.maximum(jnp.dot(xb, w1_ref[...], preferred_element_type=f32)
                    + b1_ref[...], 0.0),
        w2_ref[...], preferred_element_type=f32) + b2_ref[...]
    out_x_ref[...] = xb + fcx


def _tc_post_body(tpart_ref, stt_ref, ctt_ref, sx_ref, cx_ref,
                  wltt_ref, wlx_ref, wo_ref,
                  bo_ref, gamma_ref, beta_ref,
                  out_t_ref):
    f32 = jnp.float32
    agg_tt = stt_ref[...].astype(f32) / jnp.maximum(ctt_ref[...], 1.0)
    agg_x = sx_ref[...].astype(f32) / jnp.maximum(cx_ref[...], 1.0)
    h = (tpart_ref[...]
         + jnp.dot(agg_tt, wltt_ref[...], preferred_element_type=f32)
         + jnp.dot(agg_x, wlx_ref[...], preferred_element_type=f32))
    t2 = jnp.maximum(h, 0.0)
    o = jnp.maximum(jnp.dot(t2, wo_ref[...], preferred_element_type=f32)
                    + bo_ref[...], 0.0)
    mu = jnp.mean(o, axis=-1, keepdims=True)
    cen = o - mu
    var = jnp.mean(cen * cen, axis=-1, keepdims=True)
    ln = cen * lax.rsqrt(var + 1e-5) * gamma_ref[...] + beta_ref[...]
    out_t_ref[...] = t2 + ln


def kernel(x, t, e_t, e_xct, Wl_tt, bl_tt, Wr_tt, Wl_x, bl_x, Wr_x,
           Wo, bo, gamma, beta, W1, b1, W2, b2):
    def _prep(e):
        src = jnp.pad(e[0], (0, EP - E))
        dst = jnp.pad(e[1], (0, EP - E),
                      constant_values=N)
        return (src | (dst << DSHIFT)).reshape(NS, CPT, CHUNK)

    pk_tt = _prep(e_t)
    pk_x = _prep(e_xct)

    sum_tt, cnt_tt, sum_x, cnt_x = _sc_aggregate(
        t.astype(jnp.bfloat16), x.astype(jnp.bfloat16), pk_tt, pk_x)

    R = 2000
    row_blk = pl.BlockSpec((R, D), lambda i: (i, 0))
    cnt_blk = pl.BlockSpec((R, 1), lambda i: (i, 0))
    w_blk = pl.BlockSpec((D, D), lambda i: (0, 0))
    v_blk = pl.BlockSpec((1, D), lambda i: (0, 0))

    out_x, tpart = pl.pallas_call(
        _tc_pre_body,
        grid=(N // R,),
        in_specs=[row_blk, row_blk, w_blk, w_blk, w_blk, w_blk,
                  v_blk, v_blk, v_blk, v_blk],
        out_specs=[row_blk, row_blk],
        out_shape=[jax.ShapeDtypeStruct((N, D), jnp.float32),
                   jax.ShapeDtypeStruct((N, D), jnp.float32)],
    )(t, x, Wr_tt, Wr_x, W1, W2,
      bl_tt.reshape(1, D), bl_x.reshape(1, D),
      b1.reshape(1, D), b2.reshape(1, D))

    out_t = pl.pallas_call(
        _tc_post_body,
        grid=(N // R,),
        in_specs=[row_blk, row_blk, cnt_blk, row_blk, cnt_blk,
                  w_blk, w_blk, w_blk,
                  v_blk, v_blk, v_blk],
        out_specs=row_blk,
        out_shape=jax.ShapeDtypeStruct((N, D), jnp.float32),
    )(tpart, sum_tt, cnt_tt.reshape(NPAD, 1), sum_x, cnt_x.reshape(NPAD, 1),
      Wl_tt, Wl_x, Wo,
      bo.reshape(1, D), gamma.reshape(1, D), beta.reshape(1, D))

    return (out_x, out_t)
